# baseline (speedup 1.0000x reference)
import functools

import jax
import jax.numpy as jnp
from jax import lax
from jax.experimental import pallas as pl
from jax.experimental.pallas import tpu as pltpu
from jax.experimental.pallas import tpu_sc as plsc

N_USERS = 1000000
K = 64
BATCH = 16384
TEMPERATURE = 2.0 / 3.0
LIMIT_LOW = -0.1
LIMIT_HIGH = 1.1

_NCOLS = (N_USERS + 127) // 128
_LAST_COL = _NCOLS - 1
_LAST_BASE = _LAST_COL * 128

_info = plsc.get_sparse_core_info()
_NC, _NS = _info.num_cores, _info.num_subcores
_NW = _NC * _NS
_COLS_PER_W = 245
_IDS_BLK = 2048
_N_BLKS = BATCH // _IDS_BLK
_MAXN = BATCH + 16
_RING = 8


@functools.partial(
    pl.kernel,
    out_type=jax.ShapeDtypeStruct((BATCH, K), jnp.float32),
    mesh=plsc.VectorSubcoreMesh(core_axis_name="c", subcore_axis_name="s"),
    scratch_types=[
        pltpu.VMEM((_IDS_BLK,), jnp.int32),
        pltpu.VMEM((_MAXN,), jnp.int32),
        pltpu.VMEM((_MAXN,), jnp.int32),
        pltpu.VMEM((256,), jnp.int32),
        pltpu.VMEM((256,), jnp.int32),
        pltpu.VMEM((2, K, 128), jnp.float32),
        pltpu.VMEM((1, K), jnp.float32),
        pltpu.VMEM((_RING, K), jnp.float32),
        pltpu.VMEM((16,), jnp.int32),
        pltpu.VMEM((16,), jnp.int32),
        pltpu.SemaphoreType.DMA,
        pltpu.SemaphoreType.DMA,
        pltpu.SemaphoreType.DMA,
        pltpu.SemaphoreType.DMA,
    ],
    compiler_params=pltpu.CompilerParams(needs_layout_passes=False),
)
def _sc_gather(ids_hbm, tableT_hbm, tail_hbm, out_hbm, idsblk_v, uidl_v,
               jl_v, occ_v, occl_v, colbufs_v, tail_v, ring_v, rctr_v,
               outs_v, fsem0, fsem1, tsem, wsem):
    wid = lax.axis_index("s") * _NC + lax.axis_index("c")
    c0 = wid * _COLS_PER_W
    c1 = jnp.minimum(c0 + _COLS_PER_W, _NCOLS)
    lanes = lax.iota(jnp.int32, 16)
    ones16 = jnp.ones((16,), jnp.int32)
    zeros16 = jnp.zeros((16,), jnp.int32)
    fsems = (fsem0, fsem1)

    for t in range(16):
        occ_v[pl.ds(16 * t, 16)] = zeros16
    rctr_v[...] = zeros16
    outs_v[...] = zeros16

    def blk_body(blk, cnt_v):
        pltpu.sync_copy(ids_hbm.at[pl.ds(blk * _IDS_BLK, _IDS_BLK)], idsblk_v)

        def t_body(t, cnt_v):
            u16 = idsblk_v[pl.ds(t * 16, 16)]
            col16 = lax.shift_right_logical(u16, 7)
            m = jnp.logical_and(col16 >= c0, col16 < c1)
            j16 = blk * _IDS_BLK + t * 16 + lanes
            cnt_s = cnt_v[0]
            plsc.store_compressed(uidl_v.at[pl.ds(cnt_s, 16)], u16, mask=m)
            plsc.store_compressed(jl_v.at[pl.ds(cnt_s, 16)], j16, mask=m)
            mo = jnp.logical_and(m, col16 < _LAST_COL)
            plsc.store_scatter(occ_v, [col16 - c0], ones16, mask=mo)
            return cnt_v + plsc.all_reduce_population_count(m)

        return lax.fori_loop(0, _IDS_BLK // 16, t_body, cnt_v)

    cnt_v = lax.fori_loop(0, _N_BLKS, blk_body, zeros16)
    cnt = cnt_v[0]
    uidl_v[pl.ds(cnt, 16)] = jnp.full((16,), -1, jnp.int32)

    def ob_body(t, on_v):
        f16 = occ_v[pl.ds(16 * t, 16)]
        cols16 = c0 + 16 * t + lanes
        m = f16 > 0
        plsc.store_compressed(occl_v.at[pl.ds(on_v[0], 16)], cols16, mask=m)
        return on_v + plsc.all_reduce_population_count(m)

    occn = lax.fori_loop(0, 16, ob_body, zeros16)[0]

    def getcol(oi):
        return plsc.load_gather(occl_v, [jnp.broadcast_to(oi, (16,))])[0]

    def fetch(col, slot):
        off = pl.multiple_of(col * 128, 128)
        pltpu.async_copy(tableT_hbm.at[:, pl.ds(off, 128)],
                         colbufs_v.at[slot], fsems[slot])

    def fwait(slot):
        pltpu.make_async_copy(tableT_hbm.at[:, pl.ds(0, 128)],
                              colbufs_v.at[slot], fsems[slot]).wait()

    def wb_wait():
        pltpu.make_async_copy(ring_v.at[pl.ds(0, 1)],
                              out_hbm.at[pl.ds(0, 1)], wsem).wait()

    kvec = lanes

    def emit_row(j_l):
        r = jnp.bitwise_and(rctr_v[pl.ds(0, 16)][0], _RING - 1)
        pltpu.async_copy(ring_v.at[pl.ds(r, 1)], out_hbm.at[pl.ds(j_l, 1)],
                         wsem)
        rctr_v[...] = rctr_v[pl.ds(0, 16)] + 1
        o = outs_v[pl.ds(0, 16)][0]

        @pl.when(o + 1 > _RING)
        def _():
            wb_wait()

        outs_v[...] = jnp.broadcast_to(jnp.minimum(o + 1, _RING), (16,))

    def extract_emit(u_l, j_l, slot):
        lane = jnp.bitwise_and(u_l, 127)
        lanev = jnp.broadcast_to(lane, (16,))
        r = jnp.bitwise_and(rctr_v[pl.ds(0, 16)][0], _RING - 1)
        for q in range(4):
            vals = plsc.load_gather(colbufs_v.at[slot], [kvec + 16 * q, lanev])
            ring_v[r, pl.ds(16 * q, 16)] = vals
        emit_row(j_l)

    nch = lax.div(cnt + 15, jnp.int32(16))

    def process(col, slot):
        colv = jnp.broadcast_to(col, (16,))

        def ch_body(t, carry):
            u16 = uidl_v[pl.ds(t * 16, 16)]
            m = lax.shift_right_logical(u16, 7) == colv

            @pl.when(jnp.any(m))
            def _():
                j16 = jl_v[pl.ds(t * 16, 16)]
                for l in range(16):
                    u_l = u16[l]

                    @pl.when(lax.shift_right_logical(u_l, 7) == col)
                    def _():
                        extract_emit(u_l, j16[l], slot)

            return carry

        lax.fori_loop(0, nch, ch_body, 0)

    @pl.when(occn > 0)
    def _():
        fetch(getcol(0), 0)

    def pair_body(p, carry):
        for sl in range(2):
            oi = 2 * p + sl

            @pl.when(oi < occn)
            def _():
                @pl.when(oi + 1 < occn)
                def _():
                    fetch(getcol(oi + 1), 1 - sl)

                fwait(sl)
                process(getcol(oi), sl)

        return carry

    lax.fori_loop(0, lax.div(occn + 1, jnp.int32(2)), pair_body, 0)

    def tail_body(t, carry):
        u16 = uidl_v[pl.ds(t * 16, 16)]
        m = lax.shift_right_logical(u16, 7) == _LAST_COL

        @pl.when(jnp.any(m))
        def _():
            j16 = jl_v[pl.ds(t * 16, 16)]
            for l in range(16):
                u_l = u16[l]

                @pl.when(lax.shift_right_logical(u_l, 7) == _LAST_COL)
                def _():
                    pltpu.async_copy(tail_hbm.at[pl.ds(u_l - _LAST_BASE, 1)],
                                     tail_v, tsem).wait()
                    r = jnp.bitwise_and(rctr_v[pl.ds(0, 16)][0], _RING - 1)
                    for q in range(4):
                        ring_v[r, pl.ds(16 * q, 16)] = \
                            tail_v[0, pl.ds(16 * q, 16)]
                    emit_row(j16[l])

        return carry

    lax.fori_loop(0, nch, tail_body, 0)

    def drain_body(t, _):
        wb_wait()
        return _

    lax.fori_loop(0, outs_v[pl.ds(0, 16)][0], drain_body, 0)


_COLS_PER_BLK = 2048
_GRID = BATCH // _COLS_PER_BLK


def _tc_gate_body(a_ref, u_ref, gh_ref, z_ref, pa_ref):
    aT = jnp.transpose(a_ref[...])
    u = u_ref[...]
    pa_ref[...] = jax.nn.sigmoid(aT)
    logistic = jnp.log(u) - jnp.log(1.0 - u)
    s = jax.nn.sigmoid((logistic + aT) / TEMPERATURE)
    s_bar = s * (LIMIT_HIGH - LIMIT_LOW) + LIMIT_LOW
    z = jnp.clip(s_bar, 0.0, 1.0)
    z_ref[...] = z
    gh_ref[...] = (z > 0.5).astype(jnp.float32)


def _tc_gate(a, uT):
    ablk = pl.BlockSpec((_COLS_PER_BLK, K), lambda i: (i, 0))
    tblk = pl.BlockSpec((K, _COLS_PER_BLK), lambda i: (0, i))
    out_sds = jax.ShapeDtypeStruct((K, BATCH), jnp.float32)
    return pl.pallas_call(
        _tc_gate_body,
        grid=(_GRID,),
        in_specs=[ablk, tblk],
        out_specs=[tblk, tblk, tblk],
        out_shape=[out_sds, out_sds, out_sds],
    )(a, uT)


def kernel(user_ids, alpha, u):
    tail = lax.slice(alpha, (_LAST_BASE, 0), (N_USERS, K))
    a = _sc_gather(user_ids, alpha.T, tail)
    ghT, zT, paT = _tc_gate(a, u.T)
    return (ghT.T, zT.T, paT.T)


# speedup vs baseline: 1.6459x; 1.6459x over previous
import functools

import jax
import jax.numpy as jnp
from jax import lax
from jax.experimental import pallas as pl
from jax.experimental.pallas import tpu as pltpu
from jax.experimental.pallas import tpu_sc as plsc

N_USERS = 1000000
K = 64
BATCH = 16384
TEMPERATURE = 2.0 / 3.0
LIMIT_LOW = -0.1
LIMIT_HIGH = 1.1

_NCOLS = (N_USERS + 127) // 128
_LAST_COL = _NCOLS - 1
_LAST_BASE = _LAST_COL * 128

_info = plsc.get_sparse_core_info()
_NC, _NS = _info.num_cores, _info.num_subcores
_NW = _NC * _NS
_COLS_PER_W = 245
_IDS_BLK = 2048
_N_BLKS = BATCH // _IDS_BLK
_BCAP = 16
_MAXN = BATCH + 16
_RING = 8


@functools.partial(
    pl.kernel,
    out_type=jax.ShapeDtypeStruct((BATCH, K), jnp.float32),
    mesh=plsc.VectorSubcoreMesh(core_axis_name="c", subcore_axis_name="s"),
    scratch_types=[
        pltpu.VMEM((_IDS_BLK,), jnp.int32),
        pltpu.VMEM((_COLS_PER_W * _BCAP,), jnp.int32),
        pltpu.VMEM((_COLS_PER_W * _BCAP,), jnp.int32),
        pltpu.VMEM((256,), jnp.int32),
        pltpu.VMEM((_MAXN,), jnp.int32),
        pltpu.VMEM((_MAXN,), jnp.int32),
        pltpu.VMEM((256,), jnp.int32),
        pltpu.VMEM((256,), jnp.int32),
        pltpu.VMEM((2, K, 128), jnp.float32),
        pltpu.VMEM((1, K), jnp.float32),
        pltpu.VMEM((_RING, K), jnp.float32),
        pltpu.VMEM((16,), jnp.int32),
        pltpu.VMEM((16,), jnp.int32),
        pltpu.VMEM((16,), jnp.int32),
        pltpu.SemaphoreType.DMA,
        pltpu.SemaphoreType.DMA,
        pltpu.SemaphoreType.DMA,
        pltpu.SemaphoreType.DMA,
    ],
    compiler_params=pltpu.CompilerParams(needs_layout_passes=False),
)
def _sc_gather(ids_hbm, tableT_hbm, tail_hbm, out_hbm, idsblk_v, bu_v, bj_v,
               bcnt_v, fu_v, fj_v, occ_v, occl_v, colbufs_v, tail_v, ring_v,
               rctr_v, outs_v, fn_v, fsem0, fsem1, tsem, wsem):
    wid = lax.axis_index("s") * _NC + lax.axis_index("c")
    c0 = wid * _COLS_PER_W
    c1 = jnp.minimum(c0 + _COLS_PER_W, _NCOLS)
    lanes = lax.iota(jnp.int32, 16)
    ones16 = jnp.ones((16,), jnp.int32)
    zeros16 = jnp.zeros((16,), jnp.int32)
    fsems = (fsem0, fsem1)

    for t in range(16):
        occ_v[pl.ds(16 * t, 16)] = zeros16
        bcnt_v[pl.ds(16 * t, 16)] = zeros16
    rctr_v[...] = zeros16
    outs_v[...] = zeros16
    fn_v[...] = zeros16

    def splat(x):
        return jnp.broadcast_to(x, (16,))

    def insert(u_l, j_l):
        cl = lax.shift_right_logical(u_l, 7) - c0
        bc = plsc.load_gather(bcnt_v, [splat(cl)])[0]

        @pl.when(bc < _BCAP)
        def _():
            slot = cl * _BCAP + bc
            plsc.store_scatter(bu_v, [splat(slot)], splat(u_l))
            plsc.store_scatter(bj_v, [splat(slot)], splat(j_l))
            plsc.store_scatter(bcnt_v, [splat(cl)], splat(bc + 1))

        @pl.when(bc >= _BCAP)
        def _():
            fn = fn_v[pl.ds(0, 16)][0]
            plsc.store_scatter(fu_v, [splat(fn)], splat(u_l))
            plsc.store_scatter(fj_v, [splat(fn)], splat(j_l))
            fn_v[...] = splat(fn + 1)

    def blk_body(blk, carry):
        pltpu.sync_copy(ids_hbm.at[pl.ds(blk * _IDS_BLK, _IDS_BLK)], idsblk_v)

        def t_body(t, carry2):
            u16 = idsblk_v[pl.ds(t * 16, 16)]
            col16 = lax.shift_right_logical(u16, 7)
            m = jnp.logical_and(col16 >= c0, col16 < c1)
            mo = jnp.logical_and(m, col16 < _LAST_COL)
            plsc.store_scatter(occ_v, [col16 - c0], ones16, mask=mo)

            @pl.when(jnp.any(m))
            def _():
                jbase = blk * _IDS_BLK + t * 16
                for l in range(16):
                    u_l = u16[l]
                    colv = lax.shift_right_logical(u_l, 7)

                    @pl.when(jnp.logical_and(colv >= c0, colv < c1))
                    def _():
                        insert(u_l, jbase + l)

            return carry2

        lax.fori_loop(0, _IDS_BLK // 16, t_body, 0)
        return carry

    lax.fori_loop(0, _N_BLKS, blk_body, 0)

    def ob_body(t, on_v):
        f16 = occ_v[pl.ds(16 * t, 16)]
        cols16 = c0 + 16 * t + lanes
        m = f16 > 0
        plsc.store_compressed(occl_v.at[pl.ds(on_v[0], 16)], cols16, mask=m)
        return on_v + plsc.all_reduce_population_count(m)

    occn = lax.fori_loop(0, 16, ob_body, zeros16)[0]

    def wb_wait():
        pltpu.make_async_copy(ring_v.at[pl.ds(0, 1)],
                              out_hbm.at[pl.ds(0, 1)], wsem).wait()

    def emit_row(j_l):
        r = jnp.bitwise_and(rctr_v[pl.ds(0, 16)][0], _RING - 1)
        pltpu.async_copy(ring_v.at[pl.ds(r, 1)], out_hbm.at[pl.ds(j_l, 1)],
                         wsem)
        rctr_v[...] = rctr_v[pl.ds(0, 16)] + 1
        o = outs_v[pl.ds(0, 16)][0]

        @pl.when(o + 1 > _RING)
        def _():
            wb_wait()

        outs_v[...] = splat(jnp.minimum(o + 1, _RING))

    def extract_emit(u_l, j_l, slot):
        lanev = splat(jnp.bitwise_and(u_l, 127))
        r = jnp.bitwise_and(rctr_v[pl.ds(0, 16)][0], _RING - 1)
        for q in range(4):
            vals = plsc.load_gather(colbufs_v.at[slot], [lanes + 16 * q, lanev])
            ring_v[r, pl.ds(16 * q, 16)] = vals
        emit_row(j_l)

    def tail_emit(u_l, j_l):
        pltpu.async_copy(tail_hbm.at[pl.ds(u_l - _LAST_BASE, 1)],
                         tail_v, tsem).wait()
        r = jnp.bitwise_and(rctr_v[pl.ds(0, 16)][0], _RING - 1)
        for q in range(4):
            ring_v[r, pl.ds(16 * q, 16)] = tail_v[0, pl.ds(16 * q, 16)]
        emit_row(j_l)

    def getcol(oi):
        return plsc.load_gather(occl_v, [splat(oi)])[0]

    def fetch(col, slot):
        off = pl.multiple_of(col * 128, 128)
        pltpu.async_copy(tableT_hbm.at[:, pl.ds(off, 128)],
                         colbufs_v.at[slot], fsems[slot])

    def fwait(slot):
        pltpu.make_async_copy(tableT_hbm.at[:, pl.ds(0, 128)],
                              colbufs_v.at[slot], fsems[slot]).wait()

    def process(col, slot):
        cl = col - c0
        cntc = plsc.load_gather(bcnt_v, [splat(cl)])[0]
        u16b = bu_v[pl.ds(cl * _BCAP, 16)]
        j16b = bj_v[pl.ds(cl * _BCAP, 16)]
        for l in range(_BCAP):
            @pl.when(l < cntc)
            def _():
                extract_emit(u16b[l], j16b[l], slot)

    @pl.when(occn > 0)
    def _():
        fetch(getcol(0), 0)

    def pair_body(p, carry):
        for sl in range(2):
            oi = 2 * p + sl

            @pl.when(oi < occn)
            def _():
                @pl.when(oi + 1 < occn)
                def _():
                    fetch(getcol(oi + 1), 1 - sl)

                fwait(sl)
                process(getcol(oi), sl)

        return carry

    lax.fori_loop(0, lax.div(occn + 1, jnp.int32(2)), pair_body, 0)

    @pl.when(c1 == _NCOLS)
    def _():
        cl = jnp.int32(_LAST_COL) - c0
        cntc = plsc.load_gather(bcnt_v, [splat(cl)])[0]
        u16b = bu_v[pl.ds(cl * _BCAP, 16)]
        j16b = bj_v[pl.ds(cl * _BCAP, 16)]
        for l in range(_BCAP):
            @pl.when(l < cntc)
            def _():
                tail_emit(u16b[l], j16b[l])

    def fb_body(i, carry):
        u_l = plsc.load_gather(fu_v, [splat(i)])[0]
        j_l = plsc.load_gather(fj_v, [splat(i)])[0]
        colv = lax.shift_right_logical(u_l, 7)

        @pl.when(colv < _LAST_COL)
        def _():
            fetch(colv, 0)
            fwait(0)
            extract_emit(u_l, j_l, 0)

        @pl.when(colv >= _LAST_COL)
        def _():
            tail_emit(u_l, j_l)

        return carry

    lax.fori_loop(0, fn_v[pl.ds(0, 16)][0], fb_body, 0)

    def drain_body(t, carry):
        wb_wait()
        return carry

    lax.fori_loop(0, outs_v[pl.ds(0, 16)][0], drain_body, 0)


_COLS_PER_BLK = 2048
_GRID = BATCH // _COLS_PER_BLK


def _tc_gate_body(a_ref, u_ref, gh_ref, z_ref, pa_ref):
    aT = jnp.transpose(a_ref[...])
    u = u_ref[...]
    pa_ref[...] = jax.nn.sigmoid(aT)
    logistic = jnp.log(u) - jnp.log(1.0 - u)
    s = jax.nn.sigmoid((logistic + aT) / TEMPERATURE)
    s_bar = s * (LIMIT_HIGH - LIMIT_LOW) + LIMIT_LOW
    z = jnp.clip(s_bar, 0.0, 1.0)
    z_ref[...] = z
    gh_ref[...] = (z > 0.5).astype(jnp.float32)


def _tc_gate(a, uT):
    ablk = pl.BlockSpec((_COLS_PER_BLK, K), lambda i: (i, 0))
    tblk = pl.BlockSpec((K, _COLS_PER_BLK), lambda i: (0, i))
    out_sds = jax.ShapeDtypeStruct((K, BATCH), jnp.float32)
    return pl.pallas_call(
        _tc_gate_body,
        grid=(_GRID,),
        in_specs=[ablk, tblk],
        out_specs=[tblk, tblk, tblk],
        out_shape=[out_sds, out_sds, out_sds],
    )(a, uT)


def kernel(user_ids, alpha, u):
    tail = lax.slice(alpha, (_LAST_BASE, 0), (N_USERS, K))
    a = _sc_gather(user_ids, alpha.T, tail)
    ghT, zT, paT = _tc_gate(a, u.T)
    return (ghT.T, zT.T, paT.T)


# speedup vs baseline: 2.8418x; 1.7266x over previous
import functools

import jax
import jax.numpy as jnp
from jax import lax
from jax.experimental import pallas as pl
from jax.experimental.pallas import tpu as pltpu
from jax.experimental.pallas import tpu_sc as plsc

N_USERS = 1000000
K = 64
BATCH = 16384
TEMPERATURE = 2.0 / 3.0
LIMIT_LOW = -0.1
LIMIT_HIGH = 1.1

_NCOLS = (N_USERS + 127) // 128
_LAST_COL = _NCOLS - 1
_LAST_BASE = _LAST_COL * 128

_info = plsc.get_sparse_core_info()
_NC, _NS = _info.num_cores, _info.num_subcores
_NW = _NC * _NS
_COLS_PER_W = 245
_IDS_BLK = 2048
_N_BLKS = BATCH // _IDS_BLK
_BCAP = 16
_MAXN = BATCH + 16
_RING = 8


@functools.partial(
    pl.kernel,
    out_type=jax.ShapeDtypeStruct((BATCH, K), jnp.float32),
    mesh=plsc.VectorSubcoreMesh(core_axis_name="c", subcore_axis_name="s"),
    scratch_types=[
        pltpu.VMEM((_IDS_BLK,), jnp.int32),
        pltpu.VMEM((_MAXN,), jnp.int32),
        pltpu.VMEM((_MAXN,), jnp.int32),
        pltpu.VMEM((_COLS_PER_W * _BCAP,), jnp.int32),
        pltpu.VMEM((_COLS_PER_W * _BCAP,), jnp.int32),
        pltpu.VMEM((256,), jnp.int32),
        pltpu.VMEM((_MAXN,), jnp.int32),
        pltpu.VMEM((_MAXN,), jnp.int32),
        pltpu.VMEM((256,), jnp.int32),
        pltpu.VMEM((256,), jnp.int32),
        pltpu.VMEM((2, K, 128), jnp.float32),
        pltpu.VMEM((1, K), jnp.float32),
        pltpu.VMEM((_RING, K), jnp.float32),
        pltpu.VMEM((16,), jnp.int32),
        pltpu.VMEM((16,), jnp.int32),
        pltpu.VMEM((16,), jnp.int32),
        pltpu.SemaphoreType.DMA,
        pltpu.SemaphoreType.DMA,
        pltpu.SemaphoreType.DMA,
        pltpu.SemaphoreType.DMA,
    ],
    compiler_params=pltpu.CompilerParams(needs_layout_passes=False),
)
def _sc_gather(ids_hbm, tableT_hbm, tail_hbm, out_hbm, idsblk_v, uidl_v, jl_v,
               bu_v, bj_v, bcnt_v, fu_v, fj_v, occ_v, occl_v, colbufs_v,
               tail_v, ring_v, rctr_v, outs_v, fn_v, fsem0, fsem1, tsem, wsem):
    wid = lax.axis_index("s") * _NC + lax.axis_index("c")
    c0 = wid * _COLS_PER_W
    c1 = jnp.minimum(c0 + _COLS_PER_W, _NCOLS)
    lanes = lax.iota(jnp.int32, 16)
    ones16 = jnp.ones((16,), jnp.int32)
    zeros16 = jnp.zeros((16,), jnp.int32)
    fsems = (fsem0, fsem1)

    for t in range(16):
        occ_v[pl.ds(16 * t, 16)] = zeros16
        bcnt_v[pl.ds(16 * t, 16)] = zeros16
    rctr_v[...] = zeros16
    outs_v[...] = zeros16
    fn_v[...] = zeros16

    def splat(x):
        return jnp.broadcast_to(x, (16,))

    def insert(u_l, j_l):
        cl = lax.shift_right_logical(u_l, 7) - c0
        bc = plsc.load_gather(bcnt_v, [splat(cl)])[0]

        @pl.when(bc < _BCAP)
        def _():
            slot = cl * _BCAP + bc
            plsc.store_scatter(bu_v, [splat(slot)], splat(u_l))
            plsc.store_scatter(bj_v, [splat(slot)], splat(j_l))
            plsc.store_scatter(bcnt_v, [splat(cl)], splat(bc + 1))

        @pl.when(bc >= _BCAP)
        def _():
            fn = fn_v[pl.ds(0, 16)][0]
            plsc.store_scatter(fu_v, [splat(fn)], splat(u_l))
            plsc.store_scatter(fj_v, [splat(fn)], splat(j_l))
            fn_v[...] = splat(fn + 1)

    def blk_body(blk, cnt_v):
        pltpu.sync_copy(ids_hbm.at[pl.ds(blk * _IDS_BLK, _IDS_BLK)], idsblk_v)

        def t_body(t, cnt_v2):
            u16 = idsblk_v[pl.ds(t * 16, 16)]
            col16 = lax.shift_right_logical(u16, 7)
            m = jnp.logical_and(col16 >= c0, col16 < c1)
            mo = jnp.logical_and(m, col16 < _LAST_COL)
            plsc.store_scatter(occ_v, [col16 - c0], ones16, mask=mo)
            j16 = blk * _IDS_BLK + t * 16 + lanes
            cnt_s = cnt_v2[0]
            plsc.store_compressed(uidl_v.at[pl.ds(cnt_s, 16)], u16, mask=m)
            plsc.store_compressed(jl_v.at[pl.ds(cnt_s, 16)], j16, mask=m)
            return cnt_v2 + plsc.all_reduce_population_count(m)

        return lax.fori_loop(0, _IDS_BLK // 16, t_body, cnt_v)

    cnt = lax.fori_loop(0, _N_BLKS, blk_body, zeros16)[0]
    uidl_v[pl.ds(cnt, 16)] = jnp.full((16,), -1, jnp.int32)
    nch = lax.div(cnt + 15, jnp.int32(16))

    def a2_body(t, carry):
        u16 = uidl_v[pl.ds(t * 16, 16)]
        j16 = jl_v[pl.ds(t * 16, 16)]
        for l in range(16):
            u_l = u16[l]

            @pl.when(u_l >= 0)
            def _():
                insert(u_l, j16[l])

        return carry

    lax.fori_loop(0, nch, a2_body, 0)

    def ob_body(t, on_v):
        f16 = occ_v[pl.ds(16 * t, 16)]
        cols16 = c0 + 16 * t + lanes
        m = f16 > 0
        plsc.store_compressed(occl_v.at[pl.ds(on_v[0], 16)], cols16, mask=m)
        return on_v + plsc.all_reduce_population_count(m)

    occn = lax.fori_loop(0, 16, ob_body, zeros16)[0]

    def wb_wait():
        pltpu.make_async_copy(ring_v.at[pl.ds(0, 1)],
                              out_hbm.at[pl.ds(0, 1)], wsem).wait()

    def emit_row(j_l):
        r = jnp.bitwise_and(rctr_v[pl.ds(0, 16)][0], _RING - 1)
        pltpu.async_copy(ring_v.at[pl.ds(r, 1)], out_hbm.at[pl.ds(j_l, 1)],
                         wsem)
        rctr_v[...] = rctr_v[pl.ds(0, 16)] + 1
        o = outs_v[pl.ds(0, 16)][0]

        @pl.when(o + 1 > _RING)
        def _():
            wb_wait()

        outs_v[...] = splat(jnp.minimum(o + 1, _RING))

    def extract_emit(u_l, j_l, slot):
        lanev = splat(jnp.bitwise_and(u_l, 127))
        r = jnp.bitwise_and(rctr_v[pl.ds(0, 16)][0], _RING - 1)
        for q in range(4):
            vals = plsc.load_gather(colbufs_v.at[slot], [lanes + 16 * q, lanev])
            ring_v[r, pl.ds(16 * q, 16)] = vals
        emit_row(j_l)

    def tail_emit(u_l, j_l):
        pltpu.async_copy(tail_hbm.at[pl.ds(u_l - _LAST_BASE, 1)],
                         tail_v, tsem).wait()
        r = jnp.bitwise_and(rctr_v[pl.ds(0, 16)][0], _RING - 1)
        for q in range(4):
            ring_v[r, pl.ds(16 * q, 16)] = tail_v[0, pl.ds(16 * q, 16)]
        emit_row(j_l)

    def getcol(oi):
        return plsc.load_gather(occl_v, [splat(oi)])[0]

    def fetch(col, slot):
        off = pl.multiple_of(col * 128, 128)
        pltpu.async_copy(tableT_hbm.at[:, pl.ds(off, 128)],
                         colbufs_v.at[slot], fsems[slot])

    def fwait(slot):
        pltpu.make_async_copy(tableT_hbm.at[:, pl.ds(0, 128)],
                              colbufs_v.at[slot], fsems[slot]).wait()

    def process(col, slot):
        cl = col - c0
        cntc = plsc.load_gather(bcnt_v, [splat(cl)])[0]
        u16b = bu_v[pl.ds(cl * _BCAP, 16)]
        j16b = bj_v[pl.ds(cl * _BCAP, 16)]
        for l in range(_BCAP):
            @pl.when(l < cntc)
            def _():
                extract_emit(u16b[l], j16b[l], slot)

    @pl.when(occn > 0)
    def _():
        fetch(getcol(0), 0)

    def pair_body(p, carry):
        for sl in range(2):
            oi = 2 * p + sl

            @pl.when(oi < occn)
            def _():
                @pl.when(oi + 1 < occn)
                def _():
                    fetch(getcol(oi + 1), 1 - sl)

                fwait(sl)
                process(getcol(oi), sl)

        return carry

    lax.fori_loop(0, lax.div(occn + 1, jnp.int32(2)), pair_body, 0)

    @pl.when(c1 == _NCOLS)
    def _():
        cl = jnp.int32(_LAST_COL) - c0
        cntc = plsc.load_gather(bcnt_v, [splat(cl)])[0]
        u16b = bu_v[pl.ds(cl * _BCAP, 16)]
        j16b = bj_v[pl.ds(cl * _BCAP, 16)]
        for l in range(_BCAP):
            @pl.when(l < cntc)
            def _():
                tail_emit(u16b[l], j16b[l])

    def fb_body(i, carry):
        u_l = plsc.load_gather(fu_v, [splat(i)])[0]
        j_l = plsc.load_gather(fj_v, [splat(i)])[0]
        colv = lax.shift_right_logical(u_l, 7)

        @pl.when(colv < _LAST_COL)
        def _():
            fetch(colv, 0)
            fwait(0)
            extract_emit(u_l, j_l, 0)

        @pl.when(colv >= _LAST_COL)
        def _():
            tail_emit(u_l, j_l)

        return carry

    lax.fori_loop(0, fn_v[pl.ds(0, 16)][0], fb_body, 0)

    def drain_body(t, carry):
        wb_wait()
        return carry

    lax.fori_loop(0, outs_v[pl.ds(0, 16)][0], drain_body, 0)


_COLS_PER_BLK = 2048
_GRID = BATCH // _COLS_PER_BLK


def _tc_gate_body(a_ref, u_ref, gh_ref, z_ref, pa_ref):
    aT = jnp.transpose(a_ref[...])
    u = u_ref[...]
    pa_ref[...] = jax.nn.sigmoid(aT)
    logistic = jnp.log(u) - jnp.log(1.0 - u)
    s = jax.nn.sigmoid((logistic + aT) / TEMPERATURE)
    s_bar = s * (LIMIT_HIGH - LIMIT_LOW) + LIMIT_LOW
    z = jnp.clip(s_bar, 0.0, 1.0)
    z_ref[...] = z
    gh_ref[...] = (z > 0.5).astype(jnp.float32)


def _tc_gate(a, uT):
    ablk = pl.BlockSpec((_COLS_PER_BLK, K), lambda i: (i, 0))
    tblk = pl.BlockSpec((K, _COLS_PER_BLK), lambda i: (0, i))
    out_sds = jax.ShapeDtypeStruct((K, BATCH), jnp.float32)
    return pl.pallas_call(
        _tc_gate_body,
        grid=(_GRID,),
        in_specs=[ablk, tblk],
        out_specs=[tblk, tblk, tblk],
        out_shape=[out_sds, out_sds, out_sds],
    )(a, uT)


def kernel(user_ids, alpha, u):
    tail = lax.slice(alpha, (_LAST_BASE, 0), (N_USERS, K))
    a = _sc_gather(user_ids, alpha.T, tail)
    ghT, zT, paT = _tc_gate(a, u.T)
    return (ghT.T, zT.T, paT.T)


# speedup vs baseline: 4.8536x; 1.7079x over previous
import functools

import jax
import jax.numpy as jnp
from jax import lax
from jax.experimental import pallas as pl
from jax.experimental.pallas import tpu as pltpu
from jax.experimental.pallas import tpu_sc as plsc

N_USERS = 1000000
K = 64
BATCH = 16384
TEMPERATURE = 2.0 / 3.0
LIMIT_LOW = -0.1
LIMIT_HIGH = 1.1

_NCOLS = (N_USERS + 127) // 128
_LAST_COL = _NCOLS - 1
_LAST_BASE = _LAST_COL * 128

_info = plsc.get_sparse_core_info()
_NC, _NS = _info.num_cores, _info.num_subcores
_NW = _NC * _NS
_COLS_PER_W = 245
_IDS_BLK = 2048
_N_BLKS = BATCH // _IDS_BLK
_BCAP = 8
_MAXN = BATCH + 16
_RING = 8


@functools.partial(
    pl.kernel,
    out_type=jax.ShapeDtypeStruct((BATCH, K), jnp.float32),
    mesh=plsc.VectorSubcoreMesh(core_axis_name="c", subcore_axis_name="s"),
    scratch_types=[
        pltpu.VMEM((_IDS_BLK,), jnp.int32),
        pltpu.VMEM((_MAXN,), jnp.int32),
        pltpu.VMEM((_MAXN,), jnp.int32),
        pltpu.VMEM((_COLS_PER_W * _BCAP + 16,), jnp.int32),
        pltpu.VMEM((_COLS_PER_W * _BCAP + 16,), jnp.int32),
        pltpu.VMEM((256,), jnp.int32),
        pltpu.VMEM((_MAXN,), jnp.int32),
        pltpu.VMEM((_MAXN,), jnp.int32),
        pltpu.VMEM((256,), jnp.int32),
        pltpu.VMEM((256,), jnp.int32),
        pltpu.VMEM((4, K, 128), jnp.float32),
        pltpu.VMEM((1, K), jnp.float32),
        pltpu.VMEM((_RING, K), jnp.float32),
        pltpu.VMEM((16,), jnp.int32),
        pltpu.VMEM((16,), jnp.int32),
        pltpu.VMEM((16,), jnp.int32),
        pltpu.SemaphoreType.DMA,
        pltpu.SemaphoreType.DMA,
        pltpu.SemaphoreType.DMA,
        pltpu.SemaphoreType.DMA,
        pltpu.SemaphoreType.DMA,
        pltpu.SemaphoreType.DMA,
    ],
    compiler_params=pltpu.CompilerParams(needs_layout_passes=False),
)
def _sc_gather(ids_hbm, tableT_hbm, tail_hbm, out_hbm, idsblk_v, uidl_v, jl_v,
               bu_v, bj_v, bcnt_v, fu_v, fj_v, occ_v, occl_v, colbufs_v,
               tail_v, ring_v, rctr_v, outs_v, fn_v, fsem0, fsem1, fsem2, fsem3, tsem, wsem):
    wid = lax.axis_index("s") * _NC + lax.axis_index("c")
    c0 = wid * _COLS_PER_W
    c1 = jnp.minimum(c0 + _COLS_PER_W, _NCOLS)
    lanes = lax.iota(jnp.int32, 16)
    ones16 = jnp.ones((16,), jnp.int32)
    zeros16 = jnp.zeros((16,), jnp.int32)
    fsems = (fsem0, fsem1, fsem2, fsem3)

    for t in range(16):
        occ_v[pl.ds(16 * t, 16)] = zeros16
        bcnt_v[pl.ds(16 * t, 16)] = zeros16
    rctr_v[...] = zeros16
    outs_v[...] = zeros16
    fn_v[...] = zeros16

    def splat(x):
        return jnp.broadcast_to(x, (16,))

    def insert(u_l, j_l):
        cl = lax.shift_right_logical(u_l, 7) - c0
        bc = plsc.load_gather(bcnt_v, [splat(cl)])[0]

        @pl.when(bc < _BCAP)
        def _():
            slot = cl * _BCAP + bc
            plsc.store_scatter(bu_v, [splat(slot)], splat(u_l))
            plsc.store_scatter(bj_v, [splat(slot)], splat(j_l))
            plsc.store_scatter(bcnt_v, [splat(cl)], splat(bc + 1))

        @pl.when(bc >= _BCAP)
        def _():
            fn = fn_v[pl.ds(0, 16)][0]
            plsc.store_scatter(fu_v, [splat(fn)], splat(u_l))
            plsc.store_scatter(fj_v, [splat(fn)], splat(j_l))
            fn_v[...] = splat(fn + 1)

    def blk_body(blk, cnt_v):
        pltpu.sync_copy(ids_hbm.at[pl.ds(blk * _IDS_BLK, _IDS_BLK)], idsblk_v)

        def t_body(t, cnt_v2):
            u16 = idsblk_v[pl.ds(t * 16, 16)]
            col16 = lax.shift_right_logical(u16, 7)
            m = jnp.logical_and(col16 >= c0, col16 < c1)
            mo = jnp.logical_and(m, col16 < _LAST_COL)
            plsc.store_scatter(occ_v, [col16 - c0], ones16, mask=mo)
            j16 = blk * _IDS_BLK + t * 16 + lanes
            cnt_s = cnt_v2[0]
            plsc.store_compressed(uidl_v.at[pl.ds(cnt_s, 16)], u16, mask=m)
            plsc.store_compressed(jl_v.at[pl.ds(cnt_s, 16)], j16, mask=m)
            return cnt_v2 + plsc.all_reduce_population_count(m)

        return lax.fori_loop(0, _IDS_BLK // 16, t_body, cnt_v)

    cnt = lax.fori_loop(0, _N_BLKS, blk_body, zeros16)[0]
    uidl_v[pl.ds(cnt, 16)] = jnp.full((16,), -1, jnp.int32)
    nch = lax.div(cnt + 15, jnp.int32(16))

    def a2_body(t, carry):
        u16 = uidl_v[pl.ds(t * 16, 16)]
        j16 = jl_v[pl.ds(t * 16, 16)]
        for l in range(16):
            u_l = u16[l]

            @pl.when(u_l >= 0)
            def _():
                insert(u_l, j16[l])

        return carry

    lax.fori_loop(0, nch, a2_body, 0)

    def ob_body(t, on_v):
        f16 = occ_v[pl.ds(16 * t, 16)]
        cols16 = c0 + 16 * t + lanes
        m = f16 > 0
        plsc.store_compressed(occl_v.at[pl.ds(on_v[0], 16)], cols16, mask=m)
        return on_v + plsc.all_reduce_population_count(m)

    occn = lax.fori_loop(0, 16, ob_body, zeros16)[0]

    def wb_wait():
        pltpu.make_async_copy(ring_v.at[pl.ds(0, 1)],
                              out_hbm.at[pl.ds(0, 1)], wsem).wait()

    def emit_row(j_l):
        rc = rctr_v[pl.ds(0, 16)][0]
        r = jnp.bitwise_and(rc, _RING - 1)
        pltpu.async_copy(ring_v.at[pl.ds(r, 1)], out_hbm.at[pl.ds(j_l, 1)],
                         wsem)
        rctr_v[...] = rctr_v[pl.ds(0, 16)] + 1

        @pl.when(rc >= _RING)
        def _():
            wb_wait()

    def extract_emit(u_l, j_l, slot):
        lanev = splat(jnp.bitwise_and(u_l, 127))
        r = jnp.bitwise_and(rctr_v[pl.ds(0, 16)][0], _RING - 1)
        for q in range(4):
            vals = plsc.load_gather(colbufs_v.at[slot], [lanes + 16 * q, lanev])
            ring_v[r, pl.ds(16 * q, 16)] = vals
        emit_row(j_l)

    def tail_emit(u_l, j_l):
        pltpu.async_copy(tail_hbm.at[pl.ds(u_l - _LAST_BASE, 1)],
                         tail_v, tsem).wait()
        r = jnp.bitwise_and(rctr_v[pl.ds(0, 16)][0], _RING - 1)
        for q in range(4):
            ring_v[r, pl.ds(16 * q, 16)] = tail_v[0, pl.ds(16 * q, 16)]
        emit_row(j_l)

    def getcol(oi):
        return plsc.load_gather(occl_v, [splat(oi)])[0]

    def fetch(col, slot):
        off = pl.multiple_of(col * 128, 128)
        pltpu.async_copy(tableT_hbm.at[:, pl.ds(off, 128)],
                         colbufs_v.at[slot], fsems[slot])

    def fwait(slot):
        pltpu.make_async_copy(tableT_hbm.at[:, pl.ds(0, 128)],
                              colbufs_v.at[slot], fsems[slot]).wait()

    def process(col, slot):
        cl = col - c0
        cntc = plsc.load_gather(bcnt_v, [splat(cl)])[0]
        u16b = bu_v[pl.ds(cl * _BCAP, 16)]
        j16b = bj_v[pl.ds(cl * _BCAP, 16)]
        for l in range(_BCAP):
            @pl.when(l < cntc)
            def _():
                extract_emit(u16b[l], j16b[l], slot)

    for s0 in range(3):
        @pl.when(s0 < occn)
        def _(s0=s0):
            fetch(getcol(s0), s0)

    def quad_body(p, carry):
        for sl in range(4):
            oi = 4 * p + sl

            @pl.when(oi < occn)
            def _(sl=sl, oi=oi):
                @pl.when(oi + 3 < occn)
                def _():
                    fetch(getcol(oi + 3), (sl + 3) % 4)

                fwait(sl)
                process(getcol(oi), sl)

        return carry

    lax.fori_loop(0, lax.div(occn + 3, jnp.int32(4)), quad_body, 0)

    @pl.when(c1 == _NCOLS)
    def _():
        cl = jnp.int32(_LAST_COL) - c0
        cntc = plsc.load_gather(bcnt_v, [splat(cl)])[0]
        u16b = bu_v[pl.ds(cl * _BCAP, 16)]
        j16b = bj_v[pl.ds(cl * _BCAP, 16)]
        for l in range(_BCAP):
            @pl.when(l < cntc)
            def _():
                tail_emit(u16b[l], j16b[l])

    def fb_body(i, carry):
        u_l = plsc.load_gather(fu_v, [splat(i)])[0]
        j_l = plsc.load_gather(fj_v, [splat(i)])[0]
        colv = lax.shift_right_logical(u_l, 7)

        @pl.when(colv < _LAST_COL)
        def _():
            fetch(colv, 0)
            fwait(0)
            extract_emit(u_l, j_l, 0)

        @pl.when(colv >= _LAST_COL)
        def _():
            tail_emit(u_l, j_l)

        return carry

    lax.fori_loop(0, fn_v[pl.ds(0, 16)][0], fb_body, 0)

    def drain_body(t, carry):
        wb_wait()
        return carry

    lax.fori_loop(0, jnp.minimum(rctr_v[pl.ds(0, 16)][0], _RING),
                  drain_body, 0)


_COLS_PER_BLK = 2048
_GRID = BATCH // _COLS_PER_BLK


def _tc_gate_body(a_ref, u_ref, gh_ref, z_ref, pa_ref):
    aT = jnp.transpose(a_ref[...])
    u = u_ref[...]
    pa_ref[...] = jax.nn.sigmoid(aT)
    logistic = jnp.log(u) - jnp.log(1.0 - u)
    s = jax.nn.sigmoid((logistic + aT) / TEMPERATURE)
    s_bar = s * (LIMIT_HIGH - LIMIT_LOW) + LIMIT_LOW
    z = jnp.clip(s_bar, 0.0, 1.0)
    z_ref[...] = z
    gh_ref[...] = (z > 0.5).astype(jnp.float32)


def _tc_gate(a, uT):
    ablk = pl.BlockSpec((_COLS_PER_BLK, K), lambda i: (i, 0))
    tblk = pl.BlockSpec((K, _COLS_PER_BLK), lambda i: (0, i))
    out_sds = jax.ShapeDtypeStruct((K, BATCH), jnp.float32)
    return pl.pallas_call(
        _tc_gate_body,
        grid=(_GRID,),
        in_specs=[ablk, tblk],
        out_specs=[tblk, tblk, tblk],
        out_shape=[out_sds, out_sds, out_sds],
    )(a, uT)


def kernel(user_ids, alpha, u):
    tail = lax.slice(alpha, (_LAST_BASE, 0), (N_USERS, K))
    a = _sc_gather(user_ids, alpha.T, tail)
    ghT, zT, paT = _tc_gate(a, u.T)
    return (ghT.T, zT.T, paT.T)
